# baseline (speedup 1.0000x reference)
import functools

import jax
import jax.numpy as jnp
from jax import lax
from jax.experimental import pallas as pl
from jax.experimental.pallas import tpu as pltpu
from jax.experimental.pallas import tpu_sc as plsc

NC = 2
NS = 16
NW = NC * NS

CHUNK = 128
B_TOTAL = 1024 * 200
N_ROWS = B_TOTAL // CHUNK
ROWS_PER_W = N_ROWS // NW
EMBED = 64
NBUF = 5
N_ROUNDS = ROWS_PER_W // NBUF


def _body(table_hbm, idx_hbm, out_hbm, idx_v, *rest):
    bufs = rest[:NBUF]
    sems = rest[NBUF:]

    c = lax.axis_index("c")
    s = lax.axis_index("s")
    wid = s * NC + c
    row0 = wid * ROWS_PER_W

    pltpu.sync_copy(idx_hbm.at[wid], idx_v)

    def gather(j, b):
        pltpu.async_copy(table_hbm.at[idx_v.at[j]], bufs[b], sems[b])

    def wait(j, b):
        pltpu.make_async_copy(table_hbm.at[idx_v.at[j]], bufs[b], sems[b]).wait()

    def put(j, b):
        pltpu.sync_copy(bufs[b], out_hbm.at[pl.ds((row0 + j) * CHUNK, CHUNK)])

    for b in range(NBUF):
        gather(b, b)

    def round_body(t, carry):
        for b in range(NBUF):
            j = t * NBUF + b
            wait(j, b)
            put(j, b)
            gather(j + NBUF, b)
        return carry

    lax.fori_loop(0, N_ROUNDS - 1, round_body, 0)

    for b in range(NBUF):
        j = (N_ROUNDS - 1) * NBUF + b
        wait(j, b)
        put(j, b)


@jax.jit
def _lookup(table, idx):
    k = functools.partial(
        pl.kernel,
        out_type=jax.ShapeDtypeStruct((B_TOTAL, EMBED), jnp.float32),
        mesh=plsc.VectorSubcoreMesh(core_axis_name="c", subcore_axis_name="s"),
        scratch_types=[
            pltpu.VMEM((ROWS_PER_W, CHUNK), jnp.int32),
            *[pltpu.VMEM((CHUNK, EMBED), jnp.float32) for _ in range(NBUF)],
            *[pltpu.SemaphoreType.DMA for _ in range(NBUF)],
        ],
        compiler_params=pltpu.CompilerParams(use_tc_tiling_on_sc=False),
    )(_body)
    return k(table, idx)


def kernel(token_ids, embedding_table):
    idx = token_ids.astype(jnp.int32).reshape(NW, ROWS_PER_W, CHUNK)
    out = _lookup(embedding_table, idx)
    return out.reshape(token_ids.shape[0], token_ids.shape[1], EMBED)
